# Initial kernel scaffold; baseline (speedup 1.0000x reference)
import jax
import jax.numpy as jnp
from jax import lax
from jax.experimental import pallas as pl
from jax.experimental.pallas import tpu as pltpu
from jax.experimental.pallas import tpu_sc as plsc

N = 10000
D = 128
E = 320000
NC = 2
NS = 16
NW = NC * NS
EPT = E // NW
CH = 40
NBUF = 5
NCHUNK = EPT // CH
ROWS_PT = N // NS
HPAD = 10240

_mesh = plsc.VectorSubcoreMesh(
    core_axis_name="c", subcore_axis_name="s", num_cores=NC, num_subcores=NS)


def _hist_body(dst_hbm, out_hbm, dloc, hist):
    c = lax.axis_index("c")
    s = lax.axis_index("s")
    wid = s * NC + c
    zeros = jnp.zeros((16,), jnp.float32)

    @pl.loop(0, HPAD // 16)
    def _(i):
        hist[pl.ds(i * 16, 16)] = zeros

    pltpu.sync_copy(dst_hbm.at[pl.ds(wid * EPT, EPT)], dloc)
    ones = jnp.ones((16,), jnp.float32)

    @pl.loop(0, EPT // 16)
    def _(i):
        idx = dloc[pl.ds(i * 16, 16)]
        plsc.addupdate_scatter(hist, [idx], ones)

    pltpu.sync_copy(hist, out_hbm.at[wid])


def _sc_hist(dst):
    k = pl.kernel(
        _hist_body,
        out_type=jax.ShapeDtypeStruct((NW, HPAD), jnp.float32),
        mesh=_mesh,
        scratch_types=[
            pltpu.VMEM((EPT,), jnp.int32),
            pltpu.VMEM((HPAD,), jnp.float32),
        ],
    )
    return k(dst)


def _agg_body(g_hbm, src_hbm, dst_hbm, out_hbm, isrc, idst, rows, acc, *sems):
    gsems = sems[:NBUF]
    ssems = sems[NBUF:]
    c = lax.axis_index("c")
    s = lax.axis_index("s")
    wid = s * NC + c
    base = s * ROWS_PT
    pltpu.sync_copy(g_hbm.at[pl.ds(base, ROWS_PT)], acc.at[pl.ds(base, ROWS_PT)])
    plsc.subcore_barrier()

    ebase = wid * EPT
    for b in range(NBUF):
        pltpu.sync_copy(src_hbm.at[pl.ds(ebase + b * CH, CH)], isrc.at[b])
        pltpu.sync_copy(dst_hbm.at[pl.ds(ebase + b * CH, CH)], idst.at[b])
        pltpu.async_copy(g_hbm.at[isrc.at[b]], rows.at[b], gsems[b])

    @pl.loop(0, NCHUNK, step=NBUF)
    def _(j):
        for b in range(NBUF):
            pltpu.make_async_copy(g_hbm.at[isrc.at[b]], rows.at[b], gsems[b]).wait()
            pltpu.async_copy(rows.at[b], acc.at[idst.at[b]], ssems[b], add=True)
        for b in range(NBUF):
            nxt = j + b + NBUF

            @pl.when(nxt < NCHUNK)
            def _():
                pltpu.make_async_copy(rows.at[b], acc.at[idst.at[b]], ssems[b]).wait()
                pltpu.sync_copy(src_hbm.at[pl.ds(ebase + nxt * CH, CH)], isrc.at[b])
                pltpu.sync_copy(dst_hbm.at[pl.ds(ebase + nxt * CH, CH)], idst.at[b])
                pltpu.async_copy(g_hbm.at[isrc.at[b]], rows.at[b], gsems[b])

    for b in range(NBUF):
        pltpu.make_async_copy(rows.at[b], acc.at[idst.at[b]], ssems[b]).wait()
    plsc.subcore_barrier()
    pltpu.sync_copy(acc.at[pl.ds(base, ROWS_PT)], out_hbm.at[c, pl.ds(base, ROWS_PT)])


def _sc_aggregate(g, src, dst):
    k = pl.kernel(
        _agg_body,
        out_type=jax.ShapeDtypeStruct((NC, N, D), jnp.float32),
        mesh=_mesh,
        scratch_types=[
            pltpu.VMEM((NBUF, CH), jnp.int32),
            pltpu.VMEM((NBUF, CH), jnp.int32),
            pltpu.VMEM((NBUF, CH, D), jnp.float32),
            pltpu.VMEM_SHARED((N, D), jnp.float32),
        ] + [pltpu.SemaphoreType.DMA] * (2 * NBUF),
    )
    return k(g, src, dst)


_BK = 128
_NBK = (N + _BK - 1) // _BK


def _prep_body(x_ref, w1_ref, degp_ref, g1_ref, dinv_ref):
    deg = jnp.sum(degp_ref[...], axis=0) + 1.0
    v = lax.rsqrt(deg).reshape(1, _BK)
    dinv_blk = jnp.broadcast_to(v, (_BK, _BK)).T
    h = jnp.dot(x_ref[...], w1_ref[...], preferred_element_type=jnp.float32)
    g1_ref[...] = h * dinv_blk
    dinv_ref[...] = dinv_blk


def _tc_prep(x, W1, degp):
    degp4 = degp.reshape(NW, HPAD // _BK, 1, _BK)
    return pl.pallas_call(
        _prep_body,
        grid=(_NBK,),
        in_specs=[
            pl.BlockSpec((_BK, D), lambda i: (i, 0)),
            pl.BlockSpec((D, D), lambda i: (0, 0)),
            pl.BlockSpec((NW, 1, 1, _BK), lambda i: (0, i, 0, 0)),
        ],
        out_specs=[
            pl.BlockSpec((_BK, D), lambda i: (i, 0)),
            pl.BlockSpec((_BK, D), lambda i: (i, 0)),
        ],
        out_shape=[
            jax.ShapeDtypeStruct((N, D), jnp.float32),
            jax.ShapeDtypeStruct((N, D), jnp.float32),
        ],
    )(x, W1, degp4)


_BM = 1000


def _mid_body(p_ref, g1_ref, dinv_ref, b1_ref, w2_ref, g2_ref):
    agg = p_ref[0] + p_ref[1] - g1_ref[...]
    h1 = jnp.maximum(agg * dinv_ref[...] + b1_ref[...], 0.0)
    h2 = jnp.dot(h1, w2_ref[...], preferred_element_type=jnp.float32)
    g2_ref[...] = h2 * dinv_ref[...]


def _tc_mid(part1, g1, dinv_b, b1, W2):
    return pl.pallas_call(
        _mid_body,
        grid=(N // _BM,),
        in_specs=[
            pl.BlockSpec((NC, _BM, D), lambda i: (0, i, 0)),
            pl.BlockSpec((_BM, D), lambda i: (i, 0)),
            pl.BlockSpec((_BM, D), lambda i: (i, 0)),
            pl.BlockSpec((D,), lambda i: (0,)),
            pl.BlockSpec((D, D), lambda i: (0, 0)),
        ],
        out_specs=pl.BlockSpec((_BM, D), lambda i: (i, 0)),
        out_shape=jax.ShapeDtypeStruct((N, D), jnp.float32),
    )(part1, g1, dinv_b, b1, W2)


def _final_body(q_ref, g2_ref, dinv_ref, b2_ref, out_ref):
    agg = q_ref[0] + q_ref[1] - g2_ref[...]
    out_ref[...] = agg * dinv_ref[...] + b2_ref[...]


def _tc_final(part2, g2, dinv_b, b2):
    return pl.pallas_call(
        _final_body,
        grid=(N // _BM,),
        in_specs=[
            pl.BlockSpec((NC, _BM, D), lambda i: (0, i, 0)),
            pl.BlockSpec((_BM, D), lambda i: (i, 0)),
            pl.BlockSpec((_BM, D), lambda i: (i, 0)),
            pl.BlockSpec((D,), lambda i: (0,)),
        ],
        out_specs=pl.BlockSpec((_BM, D), lambda i: (i, 0)),
        out_shape=jax.ShapeDtypeStruct((N, D), jnp.float32),
    )(part2, g2, dinv_b, b2)


def kernel(x, edge_index, W1, b1, W2, b2):
    ei = edge_index.astype(jnp.int32)
    src = ei[0]
    dst = ei[1]
    degp = _sc_hist(dst)
    g1, dinv_b = _tc_prep(x, W1, degp)
    part1 = _sc_aggregate(g1, src, dst)
    g2 = _tc_mid(part1, g1, dinv_b, b1, W2)
    part2 = _sc_aggregate(g2, src, dst)
    return _tc_final(part2, g2, dinv_b, b2)


# speedup vs baseline: 34.2020x; 34.2020x over previous
import jax
import jax.numpy as jnp
from jax import lax
from jax.experimental import pallas as pl
from jax.experimental.pallas import tpu as pltpu
from jax.experimental.pallas import tpu_sc as plsc

N = 10000
D = 128
E = 320000
NC = 2
NS = 16
NW = NC * NS
EPT = E // NW
CH = 40
NBUF = 5
NIB = 2 * NBUF
NCHUNK = EPT // CH
ROWS_PT = 632
HPAD = 10240

import dataclasses
import functools


def _sc_compiler_params():
    cp = pltpu.CompilerParams()
    if "needs_layout_passes" in pltpu.CompilerParams.__dataclass_fields__:
        cp = dataclasses.replace(cp, needs_layout_passes=False)
    return cp


@functools.cache
def _mesh():
    return plsc.VectorSubcoreMesh(
        core_axis_name="c", subcore_axis_name="s",
        num_cores=NC, num_subcores=NS)


def _hist_body(ef_hbm, out_hbm, dloc, hist):
    c = lax.axis_index("c")
    s = lax.axis_index("s")
    wid = s * NC + c
    zeros = jnp.zeros((16,), jnp.float32)

    @pl.loop(0, HPAD // 16)
    def _(i):
        hist[pl.ds(i * 16, 16)] = zeros

    pltpu.sync_copy(ef_hbm.at[pl.ds(E + wid * EPT, EPT)], dloc)
    ones = jnp.ones((16,), jnp.float32)

    @pl.loop(0, EPT // 16)
    def _(i):
        idx = dloc[pl.ds(i * 16, 16)]
        plsc.addupdate_scatter(hist, [idx], ones)

    pltpu.sync_copy(hist, out_hbm.at[wid])


def _sc_hist(eflat):
    k = pl.kernel(
        _hist_body,
        out_type=jax.ShapeDtypeStruct((NW, HPAD), jnp.float32),
        mesh=_mesh(),
        compiler_params=_sc_compiler_params(),
        scratch_types=[
            pltpu.VMEM((EPT,), jnp.int32),
            pltpu.VMEM((HPAD,), jnp.float32),
        ],
    )
    return k(eflat)


def _agg_body(g_hbm, ef_hbm, out_hbm, iring, rows, acc, *sems):
    gsems = sems[:NBUF]
    ssems = sems[NBUF:2 * NBUF]
    isems = sems[2 * NBUF:]
    c = lax.axis_index("c")
    s = lax.axis_index("s")
    wid = s * NC + c
    base = pl.multiple_of(jnp.minimum(s * ROWS_PT, N - ROWS_PT), 8)
    ebase = wid * EPT

    def _load_idx(ck, q):
        pltpu.async_copy(ef_hbm.at[pl.ds(ebase + ck * CH, CH)],
                         iring.at[q, 0], isems[q])
        pltpu.async_copy(ef_hbm.at[pl.ds(E + ebase + ck * CH, CH)],
                         iring.at[q, 1], isems[q])

    def _wait_idx(q):
        pltpu.make_async_copy(ef_hbm.at[pl.ds(0, CH)], iring.at[q, 0],
                              isems[q]).wait()
        pltpu.make_async_copy(ef_hbm.at[pl.ds(0, CH)], iring.at[q, 1],
                              isems[q]).wait()

    for q in range(NBUF):
        _load_idx(q, q)
    pltpu.sync_copy(g_hbm.at[pl.ds(base, ROWS_PT)], acc.at[pl.ds(base, ROWS_PT)])
    plsc.subcore_barrier()

    @pl.loop(0, NCHUNK, step=NIB)
    def _(j):
        for p in range(2):
            for b in range(NBUF):
                jj = j + p * NBUF + b
                q = p * NBUF + b
                qn = (q + NBUF) % NIB

                @pl.when(jj >= NBUF)
                def _():
                    pltpu.make_async_copy(
                        rows.at[b], acc.at[iring.at[q, 1]], ssems[b]).wait()

                @pl.when(jj + NBUF < NCHUNK)
                def _():
                    _load_idx(jj + NBUF, qn)

                _wait_idx(q)
                pltpu.async_copy(g_hbm.at[iring.at[q, 0]], rows.at[b], gsems[b])
            for b in range(NBUF):
                q = p * NBUF + b
                pltpu.make_async_copy(
                    g_hbm.at[iring.at[q, 0]], rows.at[b], gsems[b]).wait()
                pltpu.async_copy(rows.at[b], acc.at[iring.at[q, 1]], ssems[b],
                                 add=True)

    for b in range(NBUF):
        pltpu.make_async_copy(rows.at[b], acc.at[iring.at[b, 1]], ssems[b]).wait()
    plsc.subcore_barrier()
    pltpu.sync_copy(acc.at[pl.ds(base, ROWS_PT)], out_hbm.at[c, pl.ds(base, ROWS_PT)])


def _sc_aggregate(g, eflat):
    k = pl.kernel(
        _agg_body,
        out_type=jax.ShapeDtypeStruct((NC, N, D), jnp.float32),
        mesh=_mesh(),
        scratch_types=[
            pltpu.VMEM((NIB, 2, CH), jnp.int32),
            pltpu.VMEM((NBUF, CH, D), jnp.float32),
            pltpu.VMEM_SHARED((N, D), jnp.float32),
        ] + [pltpu.SemaphoreType.DMA] * (2 * NBUF + NIB),
    )
    return k(g, eflat)


_BM = 1000
_BK = 128
_NBK = (N + _BK - 1) // _BK


def _mm_body(x_ref, w1_ref, h_ref):
    h_ref[...] = jnp.dot(x_ref[...], w1_ref[...],
                         preferred_element_type=jnp.float32)


def _tc_matmul(x, W1):
    return pl.pallas_call(
        _mm_body,
        grid=(N // _BM,),
        in_specs=[
            pl.BlockSpec((_BM, D), lambda i: (i, 0)),
            pl.BlockSpec((D, D), lambda i: (0, 0)),
        ],
        out_specs=pl.BlockSpec((_BM, D), lambda i: (i, 0)),
        out_shape=jax.ShapeDtypeStruct((N, D), jnp.float32),
    )(x, W1)


_BS = 1280


def _scale_body(h_ref, degp_ref, g1_ref, dinv_ref):
    deg = jnp.sum(degp_ref[...], axis=0, keepdims=True) + 1.0
    v = lax.rsqrt(deg)
    for r in range(_BS // _BK):
        vr = v[:, _BK * r:_BK * (r + 1)]
        blk = jnp.broadcast_to(vr, (_BK, _BK)).T
        sl = pl.ds(_BK * r, _BK)
        g1_ref[sl, :] = h_ref[sl, :] * blk
        dinv_ref[sl, :] = blk[:, :1]


def _tc_scale1(h1p, degp):
    return pl.pallas_call(
        _scale_body,
        grid=((N + _BS - 1) // _BS,),
        in_specs=[
            pl.BlockSpec((_BS, D), lambda i: (i, 0)),
            pl.BlockSpec((NW, _BS), lambda i: (0, i)),
        ],
        out_specs=[
            pl.BlockSpec((_BS, D), lambda i: (i, 0)),
            pl.BlockSpec((_BS, 1), lambda i: (i, 0)),
        ],
        out_shape=[
            jax.ShapeDtypeStruct((N, D), jnp.float32),
            jax.ShapeDtypeStruct((N, 1), jnp.float32),
        ],
    )(h1p, degp)


def _mid_body(p_ref, g1_ref, dinv_ref, b1_ref, w2_ref, g2_ref):
    dinv = jnp.broadcast_to(dinv_ref[...], (_BM, D))
    agg = p_ref[0] + p_ref[1] - g1_ref[...]
    h1 = jnp.maximum(agg * dinv + b1_ref[...], 0.0)
    h2 = jnp.dot(h1, w2_ref[...], preferred_element_type=jnp.float32)
    g2_ref[...] = h2 * dinv


def _tc_mid(part1, g1, dinv_c, b1, W2):
    return pl.pallas_call(
        _mid_body,
        grid=(N // _BM,),
        in_specs=[
            pl.BlockSpec((NC, _BM, D), lambda i: (0, i, 0)),
            pl.BlockSpec((_BM, D), lambda i: (i, 0)),
            pl.BlockSpec((_BM, 1), lambda i: (i, 0)),
            pl.BlockSpec((D,), lambda i: (0,)),
            pl.BlockSpec((D, D), lambda i: (0, 0)),
        ],
        out_specs=pl.BlockSpec((_BM, D), lambda i: (i, 0)),
        out_shape=jax.ShapeDtypeStruct((N, D), jnp.float32),
    )(part1, g1, dinv_c, b1, W2)


def _final_body(q_ref, g2_ref, dinv_ref, b2_ref, out_ref):
    dinv = jnp.broadcast_to(dinv_ref[...], (_BM, D))
    agg = q_ref[0] + q_ref[1] - g2_ref[...]
    out_ref[...] = agg * dinv + b2_ref[...]


def _tc_final(part2, g2, dinv_c, b2):
    return pl.pallas_call(
        _final_body,
        grid=(N // _BM,),
        in_specs=[
            pl.BlockSpec((NC, _BM, D), lambda i: (0, i, 0)),
            pl.BlockSpec((_BM, D), lambda i: (i, 0)),
            pl.BlockSpec((_BM, 1), lambda i: (i, 0)),
            pl.BlockSpec((D,), lambda i: (0,)),
        ],
        out_specs=pl.BlockSpec((_BM, D), lambda i: (i, 0)),
        out_shape=jax.ShapeDtypeStruct((N, D), jnp.float32),
    )(part2, g2, dinv_c, b2)


def kernel(x, edge_index, W1, b1, W2, b2):
    eflat = edge_index.astype(jnp.int32).reshape(2 * E)
    h1p = _tc_matmul(x, W1)
    degp = _sc_hist(eflat)
    g1, dinv_c = _tc_scale1(h1p, degp)
    part1 = _sc_aggregate(g1, eflat)
    g2 = _tc_mid(part1, g1, dinv_c, b1, W2)
    part2 = _sc_aggregate(g2, eflat)
    return _tc_final(part2, g2, dinv_c, b2)
